# baseline (speedup 1.0000x reference)
import functools

import jax
import jax.numpy as jnp
from jax import lax
from jax.experimental import pallas as pl
from jax.experimental.pallas import tpu as pltpu
from jax.experimental.pallas import tpu_sc as plsc

_B = 1024
_I = 100
_U = 100000
_PU = 16
_PUI = 4


_B_BLK = 128
_U_BLK = 8192
_NB = _B // _B_BLK
_NU = -(-_U // _U_BLK)


def _scan_body(oh_ref, idx_ref, acc_ref):
    ju = pl.program_id(1)

    @pl.when(ju == 0)
    def _init():
        acc_ref[...] = jnp.zeros_like(acc_ref)

    col = jax.lax.broadcasted_iota(jnp.int32, (1, _U_BLK), 1) + ju * _U_BLK
    contrib = jnp.where(col < _U, oh_ref[...] * col.astype(jnp.float32), 0.0)
    acc_ref[...] += jnp.sum(contrib, axis=1, keepdims=True)

    @pl.when(ju == _NU - 1)
    def _fin():
        idx_ref[...] = acc_ref[...].astype(jnp.int32)


def _user_idx_from_onehot(user_onehot):
    return pl.pallas_call(
        _scan_body,
        grid=(_NB, _NU),
        in_specs=[pl.BlockSpec((_B_BLK, _U_BLK), lambda ib, ju: (ib, ju))],
        out_specs=pl.BlockSpec((_B_BLK, 1), lambda ib, ju: (ib, 0)),
        out_shape=jax.ShapeDtypeStruct((_B, 1), jnp.int32),
        scratch_shapes=[pltpu.VMEM((_B_BLK, 1), jnp.float32)],
    )(user_onehot)


_B_BLK2 = 256
_KD = _I * _PU


def _dense_body(xu_ref, xi_ref, xint_ref, cu_ref, ci_ref, cif_ref, out_ref):
    prod = xu_ref[...] * cu_ref[...] + xi_ref[...] * ci_ref[...]
    row = jax.lax.broadcasted_iota(jnp.int32, (_KD, _I), 0)
    coli = jax.lax.broadcasted_iota(jnp.int32, (_KD, _I), 1)
    sel = (row // _PU == coli).astype(jnp.float32)
    out_ref[...] = (
        jnp.dot(prod, sel, preferred_element_type=jnp.float32,
                precision=jax.lax.Precision.HIGHEST)
        + xint_ref[...] * cif_ref[...]
    )


def _dense_util(xu2, xi2, xint2, cu_t, ci_t, cif):
    return pl.pallas_call(
        _dense_body,
        grid=(_B // _B_BLK2,),
        in_specs=[
            pl.BlockSpec((_B_BLK2, _KD), lambda ib: (ib, 0)),
            pl.BlockSpec((_B_BLK2, _KD), lambda ib: (ib, 0)),
            pl.BlockSpec((_B_BLK2, _I), lambda ib: (ib, 0)),
            pl.BlockSpec((1, _KD), lambda ib: (0, 0)),
            pl.BlockSpec((1, _KD), lambda ib: (0, 0)),
            pl.BlockSpec((1, _I), lambda ib: (0, 0)),
        ],
        out_specs=pl.BlockSpec((_B_BLK2, _I), lambda ib: (ib, 0)),
        out_shape=jax.ShapeDtypeStruct((_B, _I), jnp.float32),
    )(xu2, xi2, xint2, cu_t, ci_t, cif)


_NW = 32
_BPW = _B // _NW
_CW = _I * _PUI
_OPW = _BPW * _I
_NG = -(-_I // 16)


def _sc_body(coef_hbm, idx_hbm, xui_hbm, ud_hbm, out_hbm,
             idx_v, rows_v, x_v, ud_v, out_v, sem):
    wid = lax.axis_index("s") * 2 + lax.axis_index("c")
    base = wid * _BPW
    obase = wid * _OPW
    pltpu.sync_copy(idx_hbm.at[pl.ds(base, _BPW)], idx_v)
    gat = pltpu.async_copy(coef_hbm.at[idx_v], rows_v, sem)
    pltpu.sync_copy(xui_hbm.at[pl.ds(base, _BPW)], x_v)
    pltpu.sync_copy(ud_hbm.at[pl.ds(base, _BPW)], ud_v)
    gat.wait()

    lanes = lax.iota(jnp.int32, 16)

    def body(b, carry):
        b_vec = jnp.full((16,), b, jnp.int32)
        for k in range(_NG):
            i_vec = k * 16 + lanes
            valid = i_vec < _I
            i_cl = jnp.minimum(i_vec, _I - 1)
            acc = plsc.load_gather(ud_v, [b_vec, i_cl])
            for p in range(_PUI):
                ci = _PUI * i_cl + p
                xv = plsc.load_gather(x_v, [b_vec, ci])
                cv = plsc.load_gather(rows_v, [b_vec, ci])
                acc = acc + xv * cv
            plsc.store_scatter(out_v, [b_vec, i_cl], acc, mask=valid)
        return carry

    lax.fori_loop(0, _BPW, body, 0)
    pltpu.sync_copy(out_v, out_hbm.at[pl.ds(base, _BPW)])


def _sc_gather_util(coef_flat, user_idx, xui2, ud_flat):
    mesh = plsc.VectorSubcoreMesh(core_axis_name="c", subcore_axis_name="s")
    call = functools.partial(
        pl.kernel,
        mesh=mesh,
        out_type=jax.ShapeDtypeStruct((_B, _I), jnp.float32),
        scratch_types=[
            pltpu.VMEM((_BPW,), jnp.int32),
            pltpu.VMEM((_BPW, _CW), jnp.float32),
            pltpu.VMEM((_BPW, _CW), jnp.float32),
            pltpu.VMEM((_BPW, _I), jnp.float32),
            pltpu.VMEM((_BPW, _I), jnp.float32),
            pltpu.SemaphoreType.DMA,
        ],
        compiler_params=pltpu.CompilerParams(
            needs_layout_passes=False, use_tc_tiling_on_sc=False),
    )(_sc_body)
    return call(coef_flat, user_idx, xui2, ud_flat)


def kernel(x_u, x_i, x_ui, x_intercept, user_onehot, coef_u, coef_i, coef_ui,
           coef_intercept):
    xu2 = x_u.reshape(_B, _KD)
    xi2 = x_i.reshape(_B, _KD)
    xint2 = x_intercept.reshape(_B, _I)
    cu_t = jnp.tile(coef_u, _I).reshape(1, _KD)
    ci_t = coef_i.reshape(1, _KD)
    cif = jnp.concatenate(
        [jnp.zeros((1,), jnp.float32), coef_intercept[:, 0]]).reshape(1, _I)
    coef_flat = coef_ui.reshape(_U, _CW)
    xui2 = x_ui.reshape(_B, _CW)

    user_idx = _user_idx_from_onehot(user_onehot).reshape(_B)
    ud = _dense_util(xu2, xi2, xint2, cu_t, ci_t, cif)
    return _sc_gather_util(coef_flat, user_idx, xui2, ud)


# speedup vs baseline: 1.5634x; 1.5634x over previous
import functools

import jax
import jax.numpy as jnp
from jax import lax
from jax.experimental import pallas as pl
from jax.experimental.pallas import tpu as pltpu
from jax.experimental.pallas import tpu_sc as plsc

_B = 1024
_I = 100
_U = 100000
_PU = 16
_PUI = 4


_B_BLK = 128
_U_BLK = 8192
_NB = _B // _B_BLK
_NU = -(-_U // _U_BLK)


def _scan_body(oh_ref, idx_ref, acc_ref):
    ju = pl.program_id(1)

    @pl.when(ju == 0)
    def _init():
        acc_ref[...] = jnp.zeros_like(acc_ref)

    col = jax.lax.broadcasted_iota(jnp.int32, (1, _U_BLK), 1) + ju * _U_BLK
    contrib = jnp.where(col < _U, oh_ref[...] * col.astype(jnp.float32), 0.0)
    acc_ref[...] += jnp.sum(contrib, axis=1, keepdims=True)

    @pl.when(ju == _NU - 1)
    def _fin():
        idx_ref[...] = acc_ref[...].astype(jnp.int32)


def _user_idx_from_onehot(user_onehot):
    return pl.pallas_call(
        _scan_body,
        grid=(_NB, _NU),
        in_specs=[pl.BlockSpec((_B_BLK, _U_BLK), lambda ib, ju: (ib, ju))],
        out_specs=pl.BlockSpec((_B_BLK, 1), lambda ib, ju: (ib, 0)),
        out_shape=jax.ShapeDtypeStruct((_B, 1), jnp.int32),
        scratch_shapes=[pltpu.VMEM((_B_BLK, 1), jnp.float32)],
    )(user_onehot)


_B_BLK2 = 256
_KD = _I * _PU


def _dense_body(xu_ref, xi_ref, xint_ref, cu_ref, ci_ref, cif_ref, out_ref):
    prod = xu_ref[...] * cu_ref[...] + xi_ref[...] * ci_ref[...]
    row = jax.lax.broadcasted_iota(jnp.int32, (_KD, _I), 0)
    coli = jax.lax.broadcasted_iota(jnp.int32, (_KD, _I), 1)
    sel = (row // _PU == coli).astype(jnp.float32)
    out_ref[...] = (
        jnp.dot(prod, sel, preferred_element_type=jnp.float32,
                precision=jax.lax.Precision.HIGHEST)
        + xint_ref[...] * cif_ref[...]
    )


def _dense_util(xu2, xi2, xint2, cu_t, ci_t, cif):
    return pl.pallas_call(
        _dense_body,
        grid=(_B // _B_BLK2,),
        in_specs=[
            pl.BlockSpec((_B_BLK2, _KD), lambda ib: (ib, 0)),
            pl.BlockSpec((_B_BLK2, _KD), lambda ib: (ib, 0)),
            pl.BlockSpec((_B_BLK2, _I), lambda ib: (ib, 0)),
            pl.BlockSpec((1, _KD), lambda ib: (0, 0)),
            pl.BlockSpec((1, _KD), lambda ib: (0, 0)),
            pl.BlockSpec((1, _I), lambda ib: (0, 0)),
        ],
        out_specs=pl.BlockSpec((_B_BLK2, _I), lambda ib: (ib, 0)),
        out_shape=jax.ShapeDtypeStruct((_B, _I), jnp.float32),
    )(xu2, xi2, xint2, cu_t, ci_t, cif)


_NW = 32
_BPW = _B // _NW
_CW = _I * _PUI
_NG = -(-_I // 16)
_NCHUNK = 4
_BPC = _BPW // _NCHUNK
_ROWS_W = _BPW * _CW // 128
_ROWS_C = _ROWS_W // _NCHUNK


def _sc_body(coef_hbm, midx_hbm, lane_hbm, xui_hbm, ud_hbm, out_hbm,
             midx_v, l_v, g_v, x_v, ud_v, out_v, sem):
    wid = lax.axis_index("s") * 2 + lax.axis_index("c")
    base = wid * _BPW
    pltpu.sync_copy(midx_hbm.at[pl.ds(wid * _ROWS_W, _ROWS_W)], midx_v)
    pltpu.sync_copy(lane_hbm.at[pl.ds(wid * 2, 2)], l_v)
    pltpu.sync_copy(xui_hbm.at[pl.ds(base, _BPW)], x_v)
    pltpu.sync_copy(ud_hbm.at[pl.ds(base, _BPW)], ud_v)

    lanes = lax.iota(jnp.int32, 16)

    for c in range(_NCHUNK):
        copies = [
            pltpu.async_copy(
                coef_hbm.at[midx_v.at[c * _ROWS_C + r]], g_v.at[r], sem)
            for r in range(_ROWS_C)
        ]
        for cp in copies:
            cp.wait()

        def ub(b8, carry, c=c):
            b_loc = c * _BPC + b8
            bv = jnp.full((16,), b_loc, jnp.int32)
            lvec = plsc.load_gather(
                l_v, [jnp.full((16,), b_loc // 16, jnp.int32),
                      jnp.full((16,), b_loc % 16, jnp.int32)])
            for k in range(_NG):
                i_vec = k * 16 + lanes
                valid = i_vec < _I
                i_cl = jnp.minimum(i_vec, _I - 1)
                acc = plsc.load_gather(ud_v, [bv, i_cl])
                for p in range(_PUI):
                    j = _PUI * i_cl + p
                    t = b8 * _CW + j
                    cv = plsc.load_gather(
                        g_v, [lax.shift_right_logical(t, 7),
                              lax.bitwise_and(t, 127), lvec])
                    xv = plsc.load_gather(x_v, [bv, j])
                    acc = acc + cv * xv
                plsc.store_scatter(out_v, [bv, i_cl], acc, mask=valid)
            return carry

        lax.fori_loop(0, _BPC, ub, 0)

    pltpu.sync_copy(out_v, out_hbm.at[pl.ds(base, _BPW)])


def _sc_gather_util(coef16, midx, lane2, xui2, ud):
    mesh = plsc.VectorSubcoreMesh(core_axis_name="c", subcore_axis_name="s")
    call = functools.partial(
        pl.kernel,
        mesh=mesh,
        out_type=jax.ShapeDtypeStruct((_B, _I), jnp.float32),
        scratch_types=[
            pltpu.VMEM((_ROWS_W, 128), jnp.int32),
            pltpu.VMEM((2, 16), jnp.int32),
            pltpu.VMEM((_ROWS_C, 128, 16), jnp.float32),
            pltpu.VMEM((_BPW, _CW), jnp.float32),
            pltpu.VMEM((_BPW, _I), jnp.float32),
            pltpu.VMEM((_BPW, _I), jnp.float32),
            pltpu.SemaphoreType.DMA,
        ],
        compiler_params=pltpu.CompilerParams(
            needs_layout_passes=False, use_tc_tiling_on_sc=False),
    )(_sc_body)
    return call(coef16, midx, lane2, xui2, ud)


def kernel(x_u, x_i, x_ui, x_intercept, user_onehot, coef_u, coef_i, coef_ui,
           coef_intercept):
    xu2 = x_u.reshape(_B, _KD)
    xi2 = x_i.reshape(_B, _KD)
    xint2 = x_intercept.reshape(_B, _I)
    cu_t = jnp.tile(coef_u, _I).reshape(1, _KD)
    ci_t = coef_i.reshape(1, _KD)
    cif = jnp.concatenate(
        [jnp.zeros((1,), jnp.float32), coef_intercept[:, 0]]).reshape(1, _I)
    coef16 = jnp.transpose(coef_ui, (1, 2, 0)).reshape(_U * _CW // 16, 16)
    xui2 = x_ui.reshape(_B, _CW)

    user_idx = _user_idx_from_onehot(user_onehot).reshape(_B)
    midx = (jnp.arange(_CW, dtype=jnp.int32)[None, :] * (_U // 16)
            + (user_idx[:, None] >> 4)).reshape(_B * _CW // 128, 128)
    lane2 = (user_idx & 15).reshape(_B // 16, 16)
    ud = _dense_util(xu2, xi2, xint2, cu_t, ci_t, cif)
    return _sc_gather_util(coef16, midx, lane2, xui2, ud)


# speedup vs baseline: 2.9573x; 1.8916x over previous
import functools

import jax
import jax.numpy as jnp
from jax import lax
from jax.experimental import pallas as pl
from jax.experimental.pallas import tpu as pltpu
from jax.experimental.pallas import tpu_sc as plsc

_B = 1024
_I = 100
_U = 100000
_PU = 16
_PUI = 4


_U_BLK = 2000
_NU = _U // _U_BLK


def _scan_body(oh_ref, idx_ref, acc_ref):
    ju = pl.program_id(0)

    @pl.when(ju == 0)
    def _init():
        acc_ref[...] = jnp.zeros_like(acc_ref)

    urow = (jax.lax.broadcasted_iota(jnp.int32, (_U_BLK, 1), 0)
            + ju * _U_BLK).astype(jnp.float32)
    acc_ref[...] += jnp.sum(oh_ref[...] * urow, axis=0, keepdims=True)

    @pl.when(ju == _NU - 1)
    def _fin():
        idx_ref[...] = acc_ref[...].astype(jnp.int32)


def _user_idx_from_onehot(user_onehot):
    ohT = user_onehot.T
    out = pl.pallas_call(
        _scan_body,
        grid=(_NU,),
        in_specs=[pl.BlockSpec((_U_BLK, _B), lambda ju: (ju, 0))],
        out_specs=pl.BlockSpec((1, _B), lambda ju: (0, 0)),
        out_shape=jax.ShapeDtypeStruct((1, _B), jnp.int32),
        scratch_shapes=[pltpu.VMEM((1, _B), jnp.float32)],
    )(ohT)
    return out


_B_BLK2 = 256
_KD = _I * _PU


def _dense_body(xu_ref, xi_ref, xint_ref, cu_ref, ci_ref, cif_ref, out_ref):
    prod = xu_ref[...] * cu_ref[...] + xi_ref[...] * ci_ref[...]
    rowi = jax.lax.broadcasted_iota(jnp.int32, (_I, _KD), 0)
    colj = jax.lax.broadcasted_iota(jnp.int32, (_I, _KD), 1)
    sel = (colj // _PU == rowi).astype(jnp.float32)
    out_ref[...] = (
        jnp.dot(sel, prod, preferred_element_type=jnp.float32,
                precision=jax.lax.Precision.HIGHEST)
        + xint_ref[...] * cif_ref[...]
    )


def _dense_util(xuT, xiT, xintT, cuT, ciT, cifT):
    return pl.pallas_call(
        _dense_body,
        grid=(_B // _B_BLK2,),
        in_specs=[
            pl.BlockSpec((_KD, _B_BLK2), lambda ib: (0, ib)),
            pl.BlockSpec((_KD, _B_BLK2), lambda ib: (0, ib)),
            pl.BlockSpec((_I, _B_BLK2), lambda ib: (0, ib)),
            pl.BlockSpec((_KD, 1), lambda ib: (0, 0)),
            pl.BlockSpec((_KD, 1), lambda ib: (0, 0)),
            pl.BlockSpec((_I, 1), lambda ib: (0, 0)),
        ],
        out_specs=pl.BlockSpec((_I, _B_BLK2), lambda ib: (0, ib)),
        out_shape=jax.ShapeDtypeStruct((_I, _B), jnp.float32),
    )(xuT, xiT, xintT, cuT, ciT, cifT)


_NW = 32
_BPW = _B // _NW
_CW = _I * _PUI
_NG = -(-_I // 16)
_NCHUNK = 4
_BPC = _BPW // _NCHUNK
_ROWS_W = _BPW * _CW // 128
_ROWS_C = _ROWS_W // _NCHUNK


def _sc_body(coef_hbm, midx_hbm, lane_hbm, xui_hbm, ud_hbm, out_hbm,
             midx_v, l_v, g_v, x_v, ud_v, out_v, sem):
    wid = lax.axis_index("s") * 2 + lax.axis_index("c")
    base = wid * _BPW
    pltpu.sync_copy(midx_hbm.at[pl.ds(wid * _ROWS_W, _ROWS_W)], midx_v)
    pltpu.sync_copy(lane_hbm.at[pl.ds(wid * 2, 2)], l_v)
    pltpu.sync_copy(xui_hbm.at[pl.ds(base, _BPW)], x_v)
    pltpu.sync_copy(ud_hbm.at[pl.ds(base, _BPW)], ud_v)

    lanes = lax.iota(jnp.int32, 16)

    for c in range(_NCHUNK):
        copies = [
            pltpu.async_copy(
                coef_hbm.at[midx_v.at[c * _ROWS_C + r]], g_v.at[r], sem)
            for r in range(_ROWS_C)
        ]
        for cp in copies:
            cp.wait()

        def ub(b8, carry, c=c):
            b_loc = c * _BPC + b8
            bv = jnp.full((16,), b_loc, jnp.int32)
            lvec = plsc.load_gather(
                l_v, [jnp.full((16,), b_loc // 16, jnp.int32),
                      jnp.full((16,), b_loc % 16, jnp.int32)])
            for k in range(_NG):
                i_vec = k * 16 + lanes
                valid = i_vec < _I
                i_cl = jnp.minimum(i_vec, _I - 1)
                acc = plsc.load_gather(ud_v, [bv, i_cl])
                for p in range(_PUI):
                    j = _PUI * i_cl + p
                    t = b8 * _CW + j
                    cv = plsc.load_gather(
                        g_v, [lax.shift_right_logical(t, 7),
                              lax.bitwise_and(t, 127), lvec])
                    xv = plsc.load_gather(x_v, [bv, j])
                    acc = acc + cv * xv
                plsc.store_scatter(out_v, [bv, i_cl], acc, mask=valid)
            return carry

        lax.fori_loop(0, _BPC, ub, 0)

    pltpu.sync_copy(out_v, out_hbm.at[pl.ds(base, _BPW)])


def _sc_gather_util(coef16, midx, lane2, xui2, ud):
    mesh = plsc.VectorSubcoreMesh(core_axis_name="c", subcore_axis_name="s")
    call = functools.partial(
        pl.kernel,
        mesh=mesh,
        out_type=jax.ShapeDtypeStruct((_B, _I), jnp.float32),
        scratch_types=[
            pltpu.VMEM((_ROWS_W, 128), jnp.int32),
            pltpu.VMEM((2, 16), jnp.int32),
            pltpu.VMEM((_ROWS_C, 128, 16), jnp.float32),
            pltpu.VMEM((_BPW, _CW), jnp.float32),
            pltpu.VMEM((_BPW, _I), jnp.float32),
            pltpu.VMEM((_BPW, _I), jnp.float32),
            pltpu.SemaphoreType.DMA,
        ],
        compiler_params=pltpu.CompilerParams(
            needs_layout_passes=False, use_tc_tiling_on_sc=False),
    )(_sc_body)
    return call(coef16, midx, lane2, xui2, ud)


def kernel(x_u, x_i, x_ui, x_intercept, user_onehot, coef_u, coef_i, coef_ui,
           coef_intercept):
    xuT = jnp.transpose(x_u, (1, 2, 0)).reshape(_KD, _B)
    xiT = jnp.transpose(x_i, (1, 2, 0)).reshape(_KD, _B)
    xintT = jnp.transpose(x_intercept, (1, 2, 0)).reshape(_I, _B)
    cuT = jnp.tile(coef_u, _I).reshape(_KD, 1)
    ciT = coef_i.reshape(_KD, 1)
    cifT = jnp.concatenate(
        [jnp.zeros((1,), jnp.float32), coef_intercept[:, 0]]).reshape(_I, 1)
    coef16 = jnp.transpose(coef_ui, (1, 2, 0)).reshape(_U * _CW // 16, 16)
    xui2 = x_ui.reshape(_B, _CW)

    user_idx = _user_idx_from_onehot(user_onehot).reshape(_B)
    midx = (jnp.arange(_CW, dtype=jnp.int32)[None, :] * (_U // 16)
            + (user_idx[:, None] >> 4)).reshape(_B * _CW // 128, 128)
    lane2 = (user_idx & 15).reshape(_B // 16, 16)
    ud = _dense_util(xuT, xiT, xintT, cuT, ciT, cifT).T
    return _sc_gather_util(coef16, midx, lane2, xui2, ud)
